# baseline (speedup 1.0000x reference)
import functools

import jax
import jax.numpy as jnp
from jax import lax
from jax.experimental import pallas as pl
from jax.experimental.pallas import tpu as pltpu
from jax.experimental.pallas import tpu_sc as plsc

NC = 2
NS = 16
NW = NC * NS

CHUNK = 128
G = 4
R = G * CHUNK


def _make_gather(total, d, chunks_per_w):
    per_w = total // NW
    nsc = per_w // R

    @functools.partial(
        pl.kernel,
        out_type=jax.ShapeDtypeStruct((total, d), jnp.float32),
        mesh=plsc.VectorSubcoreMesh(core_axis_name="c", subcore_axis_name="s"),
        scratch_types=[
            pltpu.VMEM((chunks_per_w, CHUNK), jnp.int32),
            pltpu.VMEM((R, d), jnp.float32),
            pltpu.SemaphoreType.DMA,
        ],
        compiler_params=pltpu.CompilerParams(use_tc_tiling_on_sc=False),
    )
    def body(table_hbm, idx_hbm, out_hbm, idx_v, rows_v, sem):
        wid = lax.axis_index("s") * NC + lax.axis_index("c")
        pltpu.sync_copy(idx_hbm.at[wid], idx_v)
        base = wid * per_w

        @pl.loop(0, nsc)
        def _superchunk(j):
            descs = []
            for g in range(G):
                c = j * G + g
                descs.append(
                    pltpu.async_copy(
                        table_hbm.at[idx_v.at[c]],
                        rows_v.at[pl.ds(g * CHUNK, CHUNK)],
                        sem,
                    )
                )
            for desc in descs:
                desc.wait()
            pltpu.sync_copy(rows_v, out_hbm.at[pl.ds(base + j * R, R)])

    return body


def kernel(seq, embedding_weight):
    b, s = seq.shape
    _, d = embedding_weight.shape
    total = b * s
    per_w = total // NW
    chunks_per_w = per_w // CHUNK
    idx = seq.astype(jnp.int32).reshape(NW, chunks_per_w, CHUNK)
    out = _make_gather(total, d, chunks_per_w)(embedding_weight, idx)
    return out.reshape(b, s, d)


# speedup vs baseline: 1.0205x; 1.0205x over previous
import functools

import jax
import jax.numpy as jnp
from jax import lax
from jax.experimental import pallas as pl
from jax.experimental.pallas import tpu as pltpu
from jax.experimental.pallas import tpu_sc as plsc

NC = 2
NS = 16
NW = NC * NS

CHUNK = 128
G = 4
R = G * CHUNK


def _make_gather(total, d, chunks_per_w):
    per_w = total // NW
    nsc = per_w // R

    @functools.partial(
        pl.kernel,
        out_type=jax.ShapeDtypeStruct((total, d), jnp.float32),
        mesh=plsc.VectorSubcoreMesh(core_axis_name="c", subcore_axis_name="s"),
        scratch_types=[
            pltpu.VMEM((chunks_per_w, CHUNK), jnp.int32),
            pltpu.VMEM((2 * R, d), jnp.float32),
            pltpu.SemaphoreType.DMA,
            pltpu.SemaphoreType.DMA,
            pltpu.SemaphoreType.DMA,
        ],
        compiler_params=pltpu.CompilerParams(use_tc_tiling_on_sc=False),
    )
    def body(table_hbm, idx_hbm, out_hbm, idx_v, rows_v, sem_g, sem_w0, sem_w1):
        wid = lax.axis_index("s") * NC + lax.axis_index("c")
        pltpu.sync_copy(idx_hbm.at[wid], idx_v)
        base = wid * per_w
        sems_w = (sem_w0, sem_w1)

        def gather_superchunk(sc, b):
            descs = []
            for g in range(G):
                descs.append(
                    pltpu.async_copy(
                        table_hbm.at[idx_v.at[sc * G + g]],
                        rows_v.at[pl.ds(b * R + g * CHUNK, CHUNK)],
                        sem_g,
                    )
                )
            for desc in descs:
                desc.wait()

        def fire_write(sc, b):
            pltpu.async_copy(
                rows_v.at[pl.ds(b * R, R)],
                out_hbm.at[pl.ds(base + sc * R, R)],
                sems_w[b],
            )

        def wait_write(b):
            pltpu.make_async_copy(
                rows_v.at[pl.ds(b * R, R)],
                out_hbm.at[pl.ds(base, R)],
                sems_w[b],
            ).wait()

        for b in range(2):
            gather_superchunk(b, b)
            fire_write(b, b)

        @pl.loop(2, nsc, step=2)
        def _main(j):
            for b in range(2):
                wait_write(b)
                gather_superchunk(j + b, b)
                fire_write(j + b, b)

        for b in range(2):
            wait_write(b)

    return body


def kernel(seq, embedding_weight):
    b, s = seq.shape
    _, d = embedding_weight.shape
    total = b * s
    per_w = total // NW
    chunks_per_w = per_w // CHUNK
    idx = seq.astype(jnp.int32).reshape(NW, chunks_per_w, CHUNK)
    out = _make_gather(total, d, chunks_per_w)(embedding_weight, idx)
    return out.reshape(b, s, d)


# speedup vs baseline: 1.0234x; 1.0028x over previous
import functools

import jax
import jax.numpy as jnp
from jax import lax
from jax.experimental import pallas as pl
from jax.experimental.pallas import tpu as pltpu
from jax.experimental.pallas import tpu_sc as plsc

NC = 2
NS = 16
NW = NC * NS

CHUNK = 128
G = 2
R = G * CHUNK
W = 4


def _make_gather(total, d, chunks_per_w):
    per_w = total // NW
    nsc = per_w // R

    @functools.partial(
        pl.kernel,
        out_type=jax.ShapeDtypeStruct((total, d), jnp.float32),
        mesh=plsc.VectorSubcoreMesh(core_axis_name="c", subcore_axis_name="s"),
        scratch_types=(
            [pltpu.VMEM((chunks_per_w, CHUNK), jnp.int32),
             pltpu.VMEM((W * R, d), jnp.float32)]
            + [pltpu.SemaphoreType.DMA] * (2 * W)
        ),
        compiler_params=pltpu.CompilerParams(use_tc_tiling_on_sc=False),
    )
    def body(table_hbm, idx_hbm, out_hbm, idx_v, rows_v, *sems):
        wid = lax.axis_index("s") * NC + lax.axis_index("c")
        pltpu.sync_copy(idx_hbm.at[wid], idx_v)
        base = wid * per_w
        sems_g = sems[:W]
        sems_w = sems[W:]

        def fire_gather(sc, b):
            for g in range(G):
                pltpu.async_copy(
                    table_hbm.at[idx_v.at[sc * G + g]],
                    rows_v.at[pl.ds(b * R + g * CHUNK, CHUNK)],
                    sems_g[b],
                )

        def wait_gather(b):
            for g in range(G):
                pltpu.make_async_copy(
                    table_hbm.at[idx_v.at[0]],
                    rows_v.at[pl.ds(b * R + g * CHUNK, CHUNK)],
                    sems_g[b],
                ).wait()

        def fire_write(sc, b):
            pltpu.async_copy(
                rows_v.at[pl.ds(b * R, R)],
                out_hbm.at[pl.ds(base + sc * R, R)],
                sems_w[b],
            )

        def wait_write(b):
            pltpu.make_async_copy(
                rows_v.at[pl.ds(b * R, R)],
                out_hbm.at[pl.ds(base, R)],
                sems_w[b],
            ).wait()

        for b in range(W):
            fire_gather(b, b)

        @pl.loop(0, nsc - W, step=W)
        def _main(j):
            for b in range(W):
                sc = j + b
                wait_gather(b)
                fire_write(sc, b)
                wait_write(b)
                fire_gather(sc + W, b)

        for b in range(W):
            wait_gather(b)
            fire_write(nsc - W + b, b)
        for b in range(W):
            wait_write(b)

    return body


def kernel(seq, embedding_weight):
    b, s = seq.shape
    _, d = embedding_weight.shape
    total = b * s
    per_w = total // NW
    chunks_per_w = per_w // CHUNK
    idx = seq.astype(jnp.int32).reshape(NW, chunks_per_w, CHUNK)
    out = _make_gather(total, d, chunks_per_w)(embedding_weight, idx)
    return out.reshape(b, s, d)


# speedup vs baseline: 1.0248x; 1.0014x over previous
import functools

import jax
import jax.numpy as jnp
from jax import lax
from jax.experimental import pallas as pl
from jax.experimental.pallas import tpu as pltpu
from jax.experimental.pallas import tpu_sc as plsc

NC = 2
NS = 16
NW = NC * NS

CHUNK = 256
G = 1
R = G * CHUNK
W = 4


def _make_gather(total, d, chunks_per_w):
    per_w = total // NW
    nsc = per_w // R

    @functools.partial(
        pl.kernel,
        out_type=jax.ShapeDtypeStruct((total, d), jnp.float32),
        mesh=plsc.VectorSubcoreMesh(core_axis_name="c", subcore_axis_name="s"),
        scratch_types=(
            [pltpu.VMEM((chunks_per_w, CHUNK), jnp.int32),
             pltpu.VMEM((W * R, d), jnp.float32)]
            + [pltpu.SemaphoreType.DMA] * (2 * W)
        ),
        compiler_params=pltpu.CompilerParams(use_tc_tiling_on_sc=False),
    )
    def body(table_hbm, idx_hbm, out_hbm, idx_v, rows_v, *sems):
        wid = lax.axis_index("s") * NC + lax.axis_index("c")
        pltpu.sync_copy(idx_hbm.at[wid], idx_v)
        base = wid * per_w
        sems_g = sems[:W]
        sems_w = sems[W:]

        def fire_gather(sc, b):
            for g in range(G):
                pltpu.async_copy(
                    table_hbm.at[idx_v.at[sc * G + g]],
                    rows_v.at[pl.ds(b * R + g * CHUNK, CHUNK)],
                    sems_g[b],
                )

        def wait_gather(b):
            for g in range(G):
                pltpu.make_async_copy(
                    table_hbm.at[idx_v.at[0]],
                    rows_v.at[pl.ds(b * R + g * CHUNK, CHUNK)],
                    sems_g[b],
                ).wait()

        def fire_write(sc, b):
            pltpu.async_copy(
                rows_v.at[pl.ds(b * R, R)],
                out_hbm.at[pl.ds(base + sc * R, R)],
                sems_w[b],
            )

        def wait_write(b):
            pltpu.make_async_copy(
                rows_v.at[pl.ds(b * R, R)],
                out_hbm.at[pl.ds(base, R)],
                sems_w[b],
            ).wait()

        for b in range(W):
            fire_gather(b, b)

        @pl.loop(0, nsc - W, step=W)
        def _main(j):
            for b in range(W):
                sc = j + b
                wait_gather(b)
                fire_write(sc, b)
                wait_write(b)
                fire_gather(sc + W, b)

        for b in range(W):
            wait_gather(b)
            fire_write(nsc - W + b, b)
        for b in range(W):
            wait_write(b)

    return body


def kernel(seq, embedding_weight):
    b, s = seq.shape
    _, d = embedding_weight.shape
    total = b * s
    per_w = total // NW
    chunks_per_w = per_w // CHUNK
    idx = seq.astype(jnp.int32).reshape(NW, chunks_per_w, CHUNK)
    out = _make_gather(total, d, chunks_per_w)(embedding_weight, idx)
    return out.reshape(b, s, d)
